# baseline (speedup 1.0000x reference)
import functools

import jax
import jax.numpy as jnp
from jax import lax
from jax.experimental import pallas as pl
from jax.experimental.pallas import tpu as pltpu
from jax.experimental.pallas import tpu_sc as plsc

NUM_CLASSES = 20000
NQ = 5
NS = 2
D = 128
ROW = NQ * NS * D
B = 16
L = 50
T = 256
NW = 32
RPW = 32
AW = (B * L) // RPW


def _sc_gather(table, idx):
    mesh = plsc.VectorSubcoreMesh(core_axis_name="c", subcore_axis_name="s")

    @functools.partial(
        pl.kernel,
        mesh=mesh,
        out_type=jax.ShapeDtypeStruct((AW, RPW, ROW), jnp.float32),
        scratch_types=[
            pltpu.VMEM((1, RPW), jnp.int32),
            pltpu.VMEM((RPW, ROW), jnp.float32),
            pltpu.SemaphoreType.DMA,
        ],
    )
    def k(table_hbm, idx_hbm, out_hbm, idx_v, rows_v, sem):
        wid = lax.axis_index("s") * 2 + lax.axis_index("c")

        @pl.when(wid < AW)
        def _():
            pltpu.sync_copy(idx_hbm.at[wid], idx_v)
            pltpu.async_copy(table_hbm.at[idx_v.at[0]], rows_v, sem).wait()
            pltpu.sync_copy(rows_v, out_hbm.at[wid])

    return k(table, idx)


def _tc_assemble(qg, w2, loc):
    def body(qg_ref, w2_ref, loc_ref, q_ref, m_ref):
        q_ref[0] = qg_ref[0] + w2_ref[...]
        m = (loc_ref[0] != 0).astype(jnp.float32)
        m_ref[0] = jnp.broadcast_to(m[:, None, :], (L, NQ * NS, T))

    return pl.pallas_call(
        body,
        grid=(B,),
        in_specs=[
            pl.BlockSpec((1, L, ROW), lambda b: (b, 0, 0)),
            pl.BlockSpec((L, ROW), lambda b: (0, 0)),
            pl.BlockSpec((1, L, T), lambda b: (b, 0, 0)),
        ],
        out_specs=[
            pl.BlockSpec((1, L, ROW), lambda b: (b, 0, 0)),
            pl.BlockSpec((1, L, NQ * NS, T), lambda b: (b, 0, 0, 0)),
        ],
        out_shape=[
            jax.ShapeDtypeStruct((B, L, ROW), jnp.float32),
            jax.ShapeDtypeStruct((B, L, NQ * NS, T), jnp.float32),
        ],
    )(qg, w2, loc)


def kernel(batched_label_list, batched_location_map, query_bank, vision_weight):
    table = query_bank.reshape(2 * NUM_CLASSES, ROW)
    idx = (batched_label_list.astype(jnp.int32) * 2).reshape(AW, 1, RPW)
    qg = _sc_gather(table, idx)

    w2 = vision_weight[: L * NQ * NS].reshape(L, ROW)
    q, mask = _tc_assemble(qg.reshape(B, L, ROW), w2, batched_location_map)

    return (
        q.reshape(B, L * NQ * NS, D),
        mask.reshape(B, L * NQ * NS, T),
        jnp.ones((B, L), dtype=jnp.int32),
    )


# speedup vs baseline: 8.4456x; 8.4456x over previous
import functools

import jax
import jax.numpy as jnp
from jax import lax
from jax.experimental import pallas as pl
from jax.experimental.pallas import tpu as pltpu
from jax.experimental.pallas import tpu_sc as plsc

NUM_CLASSES = 20000
NQ = 5
NS = 2
D = 128
ROW = NQ * NS * D
B = 16
L = 50
T = 256
NW = 32
LPW = 32
AW = (B * L) // LPW
LPC = 16
NCHUNK = LPW // LPC


def _sc_gather(table, idx):
    mesh = plsc.VectorSubcoreMesh(core_axis_name="c", subcore_axis_name="s")

    @functools.partial(
        pl.kernel,
        mesh=mesh,
        out_type=jax.ShapeDtypeStruct((B * L, ROW), jnp.float32),
        scratch_types=[
            pltpu.VMEM((NCHUNK, 5 * LPC), jnp.int32),
            pltpu.VMEM((5 * LPC, NS, D), jnp.float32),
            pltpu.VMEM((LPW, ROW), jnp.float32),
            pltpu.SemaphoreType.DMA,
        ],
    )
    def k(table_hbm, idx_hbm, out_hbm, idx_v, rows_v, dense_v, sem):
        wid = lax.axis_index("s") * 2 + lax.axis_index("c")

        @pl.when(wid < AW)
        def _():
            pltpu.sync_copy(idx_hbm.at[wid], idx_v)
            for c in range(NCHUNK):
                pltpu.async_copy(table_hbm.at[idx_v.at[c]], rows_v, sem).wait()

                def densify(i, _):
                    row = c * LPC + i
                    for kk in range(NQ):
                        for s in range(NS):
                            for j in range(D // 16):
                                dense_v[row, pl.ds(kk * NS * D + s * D + j * 16, 16)] = (
                                    rows_v[i * NQ + kk, s, pl.ds(j * 16, 16)])
                    return 0

                lax.fori_loop(0, LPC, densify, 0)
            pltpu.sync_copy(dense_v, out_hbm.at[pl.ds(wid * LPW, LPW)])

    return k(table, idx)


def _tc_assemble(qg, w2, loc):
    def body(qg_ref, w2_ref, loc_ref, q_ref, m_ref):
        q_ref[0] = qg_ref[0] + w2_ref[...]
        m = (loc_ref[0] != 0).astype(jnp.float32)
        m_ref[0] = jnp.broadcast_to(m[:, None, :], (L, NQ * NS, T))

    return pl.pallas_call(
        body,
        grid=(B,),
        in_specs=[
            pl.BlockSpec((1, L, ROW), lambda b: (b, 0, 0)),
            pl.BlockSpec((L, ROW), lambda b: (0, 0)),
            pl.BlockSpec((1, L, T), lambda b: (b, 0, 0)),
        ],
        out_specs=[
            pl.BlockSpec((1, L, ROW), lambda b: (b, 0, 0)),
            pl.BlockSpec((1, L, NQ * NS, T), lambda b: (b, 0, 0, 0)),
        ],
        out_shape=[
            jax.ShapeDtypeStruct((B, L, ROW), jnp.float32),
            jax.ShapeDtypeStruct((B, L, NQ * NS, T), jnp.float32),
        ],
    )(qg, w2, loc)


def kernel(batched_label_list, batched_location_map, query_bank, vision_weight):
    table = query_bank.reshape(NUM_CLASSES * 10, NS, D)
    idx = (batched_label_list.astype(jnp.int32).reshape(B * L, 1) * 10
           + jnp.arange(NQ, dtype=jnp.int32)[None, :])
    idx = idx.reshape(AW, NCHUNK, 5 * LPC)
    qg = _sc_gather(table, idx)

    w2 = vision_weight[: L * NQ * NS].reshape(L, ROW)
    q, mask = _tc_assemble(qg.reshape(B, L, ROW), w2, batched_location_map)

    return (
        q.reshape(B, L * NQ * NS, D),
        mask.reshape(B, L * NQ * NS, T),
        jnp.ones((B, L), dtype=jnp.int32),
    )


# speedup vs baseline: 12.8766x; 1.5247x over previous
import functools

import jax
import jax.numpy as jnp
from jax import lax
from jax.experimental import pallas as pl
from jax.experimental.pallas import tpu as pltpu
from jax.experimental.pallas import tpu_sc as plsc

NUM_CLASSES = 20000
BANK = 10
NQ = 5
NS = 2
D = 128
ROW = NQ * NS * D
RPL = NQ * NS
B = 16
L = 50
T = 256
LPW = 32
AW = (B * L) // LPW
LPC = 16
NCHUNK = LPW // LPC
WIDX_CH = 4


def _sc_gather_add(table, vw, idx, widx):
    mesh = plsc.VectorSubcoreMesh(core_axis_name="c", subcore_axis_name="s")

    @functools.partial(
        pl.kernel,
        mesh=mesh,
        out_type=jax.ShapeDtypeStruct((B * L * RPL, D), jnp.float32),
        scratch_types=[
            pltpu.VMEM((NCHUNK, NQ * LPC), jnp.int32),
            pltpu.VMEM((WIDX_CH, 80), jnp.int32),
            pltpu.VMEM((NQ * LPC, NS, D), jnp.float32),
            pltpu.VMEM((LPW * RPL, D), jnp.float32),
            pltpu.SemaphoreType.DMA,
            pltpu.SemaphoreType.DMA,
        ],
    )
    def k(table_hbm, vw_hbm, idx_hbm, widx_hbm, out_hbm,
          idx_v, widx_v, rows_v, acc_v, sem, wsem):
        wid = lax.axis_index("s") * 2 + lax.axis_index("c")

        @pl.when(wid < AW)
        def _():
            pltpu.sync_copy(idx_hbm.at[wid], idx_v)
            pltpu.sync_copy(widx_hbm.at[wid], widx_v)
            wcopies = [
                pltpu.async_copy(vw_hbm.at[widx_v.at[c]],
                                 acc_v.at[pl.ds(c * 80, 80)], wsem)
                for c in range(WIDX_CH)
            ]
            gatherA = pltpu.async_copy(table_hbm.at[idx_v.at[0]], rows_v, sem)
            for wc in wcopies:
                wc.wait()

            def densify_add(chunk, i):
                base = (chunk * LPC + i) * RPL
                for kk in range(NQ):
                    for s in range(NS):
                        r = base + kk * NS + s
                        for j in range(D // 16):
                            sl = pl.ds(j * 16, 16)
                            acc_v[r, sl] = (acc_v[r, sl]
                                            + rows_v[i * NQ + kk, s, sl])

            gatherA.wait()
            lax.fori_loop(0, LPC, lambda i, _: (densify_add(0, i), 0)[1], 0)
            pltpu.async_copy(table_hbm.at[idx_v.at[1]], rows_v, sem).wait()
            lax.fori_loop(0, LPC, lambda i, _: (densify_add(1, i), 0)[1], 0)
            pltpu.sync_copy(acc_v,
                            out_hbm.at[pl.ds(wid * (LPW * RPL), LPW * RPL)])

    return k(table, vw, idx, widx)


def _tc_mask(loc):
    def body(loc_ref, m_ref):
        m = (loc_ref[0] != 0).astype(jnp.float32)
        m_ref[0] = jnp.broadcast_to(m[:, None, :], (L, RPL, T)).reshape(
            L * RPL, T)

    return pl.pallas_call(
        body,
        grid=(B,),
        in_specs=[pl.BlockSpec((1, L, T), lambda b: (b, 0, 0))],
        out_specs=pl.BlockSpec((1, L * RPL, T), lambda b: (b, 0, 0)),
        out_shape=jax.ShapeDtypeStruct((B, L * RPL, T), jnp.float32),
    )(loc)


def kernel(batched_label_list, batched_location_map, query_bank, vision_weight):
    table = query_bank.reshape(NUM_CLASSES * BANK, NS, D)
    vw = vision_weight[: L * RPL]

    g = jnp.arange(B * L, dtype=jnp.int32)
    idx = (batched_label_list.astype(jnp.int32).reshape(B * L, 1) * BANK
           + jnp.arange(NQ, dtype=jnp.int32)[None, :])
    idx = idx.reshape(AW, NCHUNK, NQ * LPC)
    widx = ((g % L).reshape(B * L, 1) * RPL
            + jnp.arange(RPL, dtype=jnp.int32)[None, :])
    widx = widx.reshape(AW, WIDX_CH, 80)

    q2d = _sc_gather_add(table, vw, idx, widx)
    mask = _tc_mask(batched_location_map)

    return (
        q2d.reshape(B, L * RPL, D),
        mask,
        jnp.ones((B, L), dtype=jnp.int32),
    )


# speedup vs baseline: 14.3756x; 1.1164x over previous
import functools

import jax
import jax.numpy as jnp
from jax import lax
from jax.experimental import pallas as pl
from jax.experimental.pallas import tpu as pltpu
from jax.experimental.pallas import tpu_sc as plsc

NUM_CLASSES = 20000
BANK = 10
NQ = 5
NS = 2
D = 128
RPL = NQ * NS
B = 16
L = 50
T = 256
LPW = 2
AW = L // LPW
SLAB = LPW * RPL * B
WIDX_CH = 4


def _sc_gather_add(table, vw, labT, widx):
    mesh = plsc.VectorSubcoreMesh(core_axis_name="c", subcore_axis_name="s")

    @functools.partial(
        pl.kernel,
        mesh=mesh,
        out_type=jax.ShapeDtypeStruct((L * RPL * B, D), jnp.float32),
        scratch_types=[
            pltpu.VMEM((LPW, B), jnp.int32),
            pltpu.VMEM((LPW, NQ * B), jnp.int32),
            pltpu.VMEM((WIDX_CH, 80), jnp.int32),
            pltpu.VMEM((NQ * B, NS, D), jnp.float32),
            pltpu.VMEM((SLAB, D), jnp.float32),
            pltpu.SemaphoreType.DMA,
            pltpu.SemaphoreType.DMA,
        ],
    )
    def k(table_hbm, vw_hbm, labT_hbm, widx_hbm, out_hbm,
          lab_v, idx_v, widx_v, rows_v, acc_v, sem, wsem):
        wid = lax.axis_index("s") * 2 + lax.axis_index("c")

        @pl.when(wid < AW)
        def _():
            pltpu.sync_copy(labT_hbm.at[wid], lab_v)
            pltpu.sync_copy(widx_hbm.at[wid], widx_v)
            wcopies = [
                pltpu.async_copy(vw_hbm.at[widx_v.at[c]],
                                 acc_v.at[pl.ds(c * 80, 80)], wsem)
                for c in range(WIDX_CH)
            ]
            for c in range(LPW):
                lv = lab_v[c] * BANK
                for kk in range(NQ):
                    idx_v[c, pl.ds(kk * B, B)] = lv + kk
            gatherA = pltpu.async_copy(table_hbm.at[idx_v.at[0]], rows_v, sem)
            for wc in wcopies:
                wc.wait()

            def densify_add(chunk, i):
                for kk in range(NQ):
                    for s in range(NS):
                        r = (chunk * RPL + kk * NS + s) * B + i
                        for j in range(D // 16):
                            sl = pl.ds(j * 16, 16)
                            acc_v[r, sl] = (acc_v[r, sl]
                                            + rows_v[kk * B + i, s, sl])

            gatherA.wait()
            lax.fori_loop(0, B, lambda i, _: (densify_add(0, i), 0)[1], 0)
            pltpu.async_copy(table_hbm.at[idx_v.at[1]], rows_v, sem).wait()
            lax.fori_loop(0, B, lambda i, _: (densify_add(1, i), 0)[1], 0)
            pltpu.sync_copy(acc_v, out_hbm.at[pl.ds(wid * SLAB, SLAB)])

    return k(table, vw, labT, widx)


def _tc_mask(locT):
    def body(loc_ref, m_ref):
        m = (loc_ref[...] != 0).astype(jnp.float32)
        m_ref[...] = jnp.broadcast_to(m[:, None, :, :],
                                      (L, RPL, B, T)).reshape(L * RPL, B, T)

    return pl.pallas_call(
        body,
        grid=(1,),
        in_specs=[pl.BlockSpec((L, B, T), lambda g: (0, 0, 0))],
        out_specs=pl.BlockSpec((L * RPL, B, T), lambda g: (0, 0, 0)),
        out_shape=jax.ShapeDtypeStruct((L * RPL, B, T), jnp.float32),
    )(locT)


def kernel(batched_label_list, batched_location_map, query_bank, vision_weight):
    table = query_bank.reshape(NUM_CLASSES * BANK, NS, D)

    labT = batched_label_list.astype(jnp.int32).T.reshape(AW, LPW, B)
    widx = (jnp.arange(L * RPL * B, dtype=jnp.int32) // B).reshape(
        AW, WIDX_CH, 80)

    q2d = _sc_gather_add(table, vision_weight, labT, widx)
    locT = batched_location_map.transpose(1, 0, 2)
    maskT = _tc_mask(locT)

    return (
        q2d.reshape(L * RPL, B, D).transpose(1, 0, 2),
        maskT.transpose(1, 0, 2),
        jnp.ones((B, L), dtype=jnp.int32),
    )


# speedup vs baseline: 15.6667x; 1.0898x over previous
import functools

import jax
import jax.numpy as jnp
from jax import lax
from jax.experimental import pallas as pl
from jax.experimental.pallas import tpu as pltpu
from jax.experimental.pallas import tpu_sc as plsc

NUM_CLASSES = 20000
BANK = 10
NQ = 5
NS = 2
D = 128
RPL = NQ * NS
B = 16
L = 50
T = 256
LPW = 2
AW = L // LPW
SLAB = LPW * RPL * B


def _sc_gather_add(table, vw, labT, widx):
    mesh = plsc.VectorSubcoreMesh(core_axis_name="c", subcore_axis_name="s")

    @functools.partial(
        pl.kernel,
        mesh=mesh,
        out_type=jax.ShapeDtypeStruct((L * RPL * B, D), jnp.float32),
        scratch_types=[
            pltpu.VMEM((LPW, B), jnp.int32),
            pltpu.VMEM((LPW, NQ * B), jnp.int32),
            pltpu.VMEM((1, 32), jnp.int32),
            pltpu.VMEM((NQ * B, NS, D), jnp.float32),
            pltpu.VMEM((32, D), jnp.float32),
            pltpu.VMEM((SLAB, D), jnp.float32),
            pltpu.SemaphoreType.DMA,
            pltpu.SemaphoreType.DMA,
        ],
    )
    def k(table_hbm, vw_hbm, labT_hbm, widx_hbm, out_hbm,
          lab_v, idx_v, widx_v, rows_v, wrows_v, acc_v, sem, wsem):
        wid = lax.axis_index("s") * 2 + lax.axis_index("c")

        @pl.when(wid < AW)
        def _():
            pltpu.sync_copy(labT_hbm.at[wid], lab_v)
            pltpu.sync_copy(widx_hbm.at[wid], widx_v)
            wcopy = pltpu.async_copy(vw_hbm.at[widx_v.at[0]], wrows_v, wsem)
            for c in range(LPW):
                lv = lab_v[c] * BANK
                for kk in range(NQ):
                    idx_v[c, pl.ds(kk * B, B)] = lv + kk
            gatherA = pltpu.async_copy(table_hbm.at[idx_v.at[0]], rows_v, sem)
            wcopy.wait()

            def densify_add(chunk, i):
                for kk in range(NQ):
                    for s in range(NS):
                        lt = chunk * RPL + kk * NS + s
                        r = lt * B + i
                        for j in range(D // 16):
                            sl = pl.ds(j * 16, 16)
                            acc_v[r, sl] = (wrows_v[lt, sl]
                                            + rows_v[kk * B + i, s, sl])

            gatherA.wait()
            lax.fori_loop(0, B, lambda i, _: (densify_add(0, i), 0)[1], 0)
            pltpu.async_copy(table_hbm.at[idx_v.at[1]], rows_v, sem).wait()
            lax.fori_loop(0, B, lambda i, _: (densify_add(1, i), 0)[1], 0)
            pltpu.sync_copy(acc_v, out_hbm.at[pl.ds(wid * SLAB, SLAB)])

    return k(table, vw, labT, widx)


def _tc_mask(locT):
    def body(loc_ref, m_ref):
        m = (loc_ref[...] != 0).astype(jnp.float32)
        m_ref[...] = jnp.broadcast_to(m[:, None, :, :],
                                      (L, RPL, B, T)).reshape(L * RPL, B, T)

    return pl.pallas_call(
        body,
        grid=(1,),
        in_specs=[pl.BlockSpec((L, B, T), lambda g: (0, 0, 0))],
        out_specs=pl.BlockSpec((L * RPL, B, T), lambda g: (0, 0, 0)),
        out_shape=jax.ShapeDtypeStruct((L * RPL, B, T), jnp.float32),
    )(locT)


def kernel(batched_label_list, batched_location_map, query_bank, vision_weight):
    table = query_bank.reshape(NUM_CLASSES * BANK, NS, D)

    labT = batched_label_list.astype(jnp.int32).T.reshape(AW, LPW, B)
    widx = (jnp.arange(AW, dtype=jnp.int32)[:, None] * (LPW * RPL)
            + jnp.arange(32, dtype=jnp.int32)[None, :] % (LPW * RPL))
    widx = widx.reshape(AW, 1, 32)

    q2d = _sc_gather_add(table, vision_weight, labT, widx)
    locT = batched_location_map.transpose(1, 0, 2)
    maskT = _tc_mask(locT)

    return (
        q2d.reshape(L * RPL, B, D).transpose(1, 0, 2),
        maskT.transpose(1, 0, 2),
        jnp.ones((B, L), dtype=jnp.int32),
    )


# speedup vs baseline: 15.8287x; 1.0103x over previous
import functools

import jax
import jax.numpy as jnp
from jax import lax
from jax.experimental import pallas as pl
from jax.experimental.pallas import tpu as pltpu
from jax.experimental.pallas import tpu_sc as plsc

NUM_CLASSES = 20000
BANK = 10
NQ = 5
NS = 2
D = 128
RPL = NQ * NS
B = 16
L = 50
T = 256
LPW = 2
AW = L // LPW
SLAB = LPW * RPL * B


def _sc_gather_add(table, vw, labT, widx):
    mesh = plsc.VectorSubcoreMesh(core_axis_name="c", subcore_axis_name="s")

    @functools.partial(
        pl.kernel,
        mesh=mesh,
        out_type=jax.ShapeDtypeStruct((L * RPL * B, D), jnp.float32),
        scratch_types=[
            pltpu.VMEM((LPW, B), jnp.int32),
            pltpu.VMEM((NQ, LPW * B), jnp.int32),
            pltpu.VMEM((1, 32), jnp.int32),
            pltpu.VMEM((LPW * B, NS, D), jnp.float32),
            pltpu.VMEM((LPW * B, NS, D), jnp.float32),
            pltpu.VMEM((32, D), jnp.float32),
            pltpu.VMEM((SLAB, D), jnp.float32),
            pltpu.SemaphoreType.DMA,
            pltpu.SemaphoreType.DMA,
            pltpu.SemaphoreType.DMA,
        ],
    )
    def k(table_hbm, vw_hbm, labT_hbm, widx_hbm, out_hbm,
          lab_v, idx_v, widx_v, rows_a, rows_b, wrows_v, acc_v,
          sem_a, sem_b, wsem):
        wid = lax.axis_index("s") * 2 + lax.axis_index("c")

        @pl.when(wid < AW)
        def _():
            pltpu.sync_copy(labT_hbm.at[wid], lab_v)
            pltpu.sync_copy(widx_hbm.at[wid], widx_v)
            wcopy = pltpu.async_copy(vw_hbm.at[widx_v.at[0]], wrows_v, wsem)
            for kk in range(NQ):
                for c in range(LPW):
                    idx_v[kk, pl.ds(c * B, B)] = lab_v[c] * BANK + kk
            bufs = (rows_a, rows_b)
            sems = (sem_a, sem_b)

            def gather(kk):
                return pltpu.async_copy(table_hbm.at[idx_v.at[kk]],
                                        bufs[kk % 2], sems[kk % 2])

            def densify_add(kk, i):
                rows = bufs[kk % 2]
                for c in range(LPW):
                    for s in range(NS):
                        lt = c * RPL + kk * NS + s
                        for j in range(D // 16):
                            sl = pl.ds(j * 16, 16)
                            acc_v[lt * B + i, sl] = (
                                wrows_v[lt, sl] + rows[c * B + i, s, sl])

            copies = [gather(0), gather(1)]
            wcopy.wait()
            for kk in range(NQ):
                copies[kk].wait()
                lax.fori_loop(
                    0, B, lambda i, _, kk=kk: (densify_add(kk, i), 0)[1], 0)
                if kk + 2 < NQ:
                    copies.append(gather(kk + 2))
            pltpu.sync_copy(acc_v, out_hbm.at[pl.ds(wid * SLAB, SLAB)])

    return k(table, vw, labT, widx)


def _tc_mask(locT):
    def body(loc_ref, m_ref):
        m = (loc_ref[...] != 0).astype(jnp.float32)
        m_ref[...] = jnp.broadcast_to(m[:, None, :, :],
                                      (L, RPL, B, T)).reshape(L * RPL, B, T)

    return pl.pallas_call(
        body,
        grid=(1,),
        in_specs=[pl.BlockSpec((L, B, T), lambda g: (0, 0, 0))],
        out_specs=pl.BlockSpec((L * RPL, B, T), lambda g: (0, 0, 0)),
        out_shape=jax.ShapeDtypeStruct((L * RPL, B, T), jnp.float32),
    )(locT)


def kernel(batched_label_list, batched_location_map, query_bank, vision_weight):
    table = query_bank.reshape(NUM_CLASSES * BANK, NS, D)

    labT = batched_label_list.astype(jnp.int32).T.reshape(AW, LPW, B)
    widx = (jnp.arange(AW, dtype=jnp.int32)[:, None] * (LPW * RPL)
            + jnp.arange(32, dtype=jnp.int32)[None, :] % (LPW * RPL))
    widx = widx.reshape(AW, 1, 32)

    q2d = _sc_gather_add(table, vision_weight, labT, widx)
    locT = batched_location_map.transpose(1, 0, 2)
    maskT = _tc_mask(locT)

    return (
        q2d.reshape(L * RPL, B, D).transpose(1, 0, 2),
        maskT.transpose(1, 0, 2),
        jnp.ones((B, L), dtype=jnp.int32),
    )
